# baseline (speedup 1.0000x reference)
import functools

import jax
import jax.numpy as jnp
from jax import lax
from jax.experimental import pallas as pl
from jax.experimental.pallas import tpu as pltpu
from jax.experimental.pallas import tpu_sc as plsc

N = 10000
NPAD = 10112
E = 320000
NG = 64
NC, NS = 2, 16
NW = NC * NS
CHUNK = 128
NCHUNK = 80
EPW = NCHUNK * CHUNK
EPAD = NW * EPW
ROWS_PER_TILE = NPAD // NS

@functools.lru_cache(maxsize=None)
def _get_mesh():
  return plsc.VectorSubcoreMesh(
      core_axis_name="c", subcore_axis_name="s", num_cores=NC, num_subcores=NS)


def _deg_body(dst_hbm, zeros_hbm, ones_hbm, out_hbm, didx, ones_v, acc, sem):
  c = lax.axis_index("c")
  s = lax.axis_index("s")
  w = s * NC + c
  rows = pl.ds(s * ROWS_PER_TILE, ROWS_PER_TILE)
  pltpu.sync_copy(zeros_hbm.at[rows], acc.at[rows])
  pltpu.sync_copy(dst_hbm.at[w], didx)
  pltpu.sync_copy(ones_hbm, ones_v)
  plsc.subcore_barrier()

  def body(j, carry):
    pltpu.sync_copy(ones_v, acc.at[didx.at[j]], add=True)
    return carry

  lax.fori_loop(0, NCHUNK, body, 0)
  plsc.subcore_barrier()
  pltpu.sync_copy(acc.at[rows], out_hbm.at[c, rows])


@functools.lru_cache(maxsize=None)
def _deg_call():
  return pl.kernel(
      _deg_body,
      out_type=jax.ShapeDtypeStruct((NC, NPAD, 16), jnp.float32),
      mesh=_get_mesh(),
      compiler_params=pltpu.CompilerParams(use_tc_tiling_on_sc=False),
      scratch_types=[
          pltpu.VMEM((NCHUNK, CHUNK), jnp.int32),
          pltpu.VMEM((CHUNK, 16), jnp.float32),
          pltpu.VMEM_SHARED((NPAD, 16), jnp.float32),
          pltpu.SemaphoreType.DMA,
      ],
  )


def _sc_deg(dst_t, zeros16, ones16):
  return _deg_call()(dst_t, zeros16, ones16)


NBUF = 4


def _seg_body(g_hbm, src_hbm, dst_hbm, zeros_hbm, out_hbm,
              sidx, didx, rows_bufs, acc, gsems, ssems):
  c = lax.axis_index("c")
  s = lax.axis_index("s")
  w = s * NC + c
  rows = pl.ds(s * ROWS_PER_TILE, ROWS_PER_TILE)
  pltpu.sync_copy(zeros_hbm.at[rows], acc.at[rows])
  pltpu.sync_copy(src_hbm.at[w], sidx)
  pltpu.sync_copy(dst_hbm.at[w], didx)
  plsc.subcore_barrier()

  for b in range(NBUF):
    pltpu.async_copy(g_hbm.at[sidx.at[b]], rows_bufs[b], gsems[b])

  def body(i, carry):
    j0 = NBUF * i
    for b in range(NBUF):
      pltpu.make_async_copy(g_hbm.at[sidx.at[0]], rows_bufs[b],
                            gsems[b]).wait()
      pltpu.async_copy(rows_bufs[b], acc.at[didx.at[j0 + b]], ssems[b],
                       add=True)
    for b in range(NBUF):
      pltpu.make_async_copy(rows_bufs[b], acc.at[didx.at[0]],
                            ssems[b]).wait()
      pltpu.async_copy(g_hbm.at[sidx.at[j0 + NBUF + b]], rows_bufs[b],
                       gsems[b])
    return carry

  lax.fori_loop(0, NCHUNK // NBUF - 1, body, 0)
  j0 = NCHUNK - NBUF
  for b in range(NBUF):
    pltpu.make_async_copy(g_hbm.at[sidx.at[0]], rows_bufs[b],
                          gsems[b]).wait()
    pltpu.async_copy(rows_bufs[b], acc.at[didx.at[j0 + b]], ssems[b],
                     add=True)
  for b in range(NBUF):
    pltpu.make_async_copy(rows_bufs[b], acc.at[didx.at[0]],
                          ssems[b]).wait()
  plsc.subcore_barrier()
  pltpu.sync_copy(acc.at[rows], out_hbm.at[c, rows])


@functools.lru_cache(maxsize=None)
def _seg_call(width):
  return pl.kernel(
      _seg_body,
      out_type=jax.ShapeDtypeStruct((NC, NPAD, width), jnp.float32),
      mesh=_get_mesh(),
      compiler_params=pltpu.CompilerParams(use_tc_tiling_on_sc=False),
      scratch_types=[
          pltpu.VMEM((NCHUNK, CHUNK), jnp.int32),
          pltpu.VMEM((NCHUNK, CHUNK), jnp.int32),
          [pltpu.VMEM((CHUNK, width), jnp.float32) for _ in range(NBUF)],
          pltpu.VMEM_SHARED((NPAD, width), jnp.float32),
          [pltpu.SemaphoreType.DMA for _ in range(NBUF)],
          [pltpu.SemaphoreType.DMA for _ in range(NBUF)],
      ],
  )


def _sc_seg(g, src_t, dst_t, zeros, width):
  return _seg_call(width)(g, src_t, dst_t, zeros)


def _tc_g1_body(x_ref, w1_ref, deg_ref, g1_ref):
  deg = deg_ref[0, :, 0:1] + deg_ref[1, :, 0:1] + 1.0
  dinv = lax.rsqrt(deg)
  p = jnp.dot(x_ref[...], w1_ref[...], preferred_element_type=jnp.float32)
  g1_ref[...] = dinv * p


def _tc_layer_body(s_ref, g_ref, deg_ref, b_ref, wn_ref, h_ref, gn_ref):
  deg = deg_ref[0, :, 0:1] + deg_ref[1, :, 0:1] + 1.0
  dinv = lax.rsqrt(deg)
  tot = s_ref[0] + s_ref[1] + g_ref[...]
  h = jnp.tanh(dinv * tot + b_ref[...])
  row = lax.broadcasted_iota(jnp.int32, (NPAD, 1), 0)
  h = jnp.where(row < N, h, 0.0)
  h_ref[...] = h
  gn_ref[...] = dinv * jnp.dot(h, wn_ref[...],
                               preferred_element_type=jnp.float32)


def _tc_head_body(s4_ref, g4_ref, deg_ref, b4_ref, h1_ref, h2_ref, h3_ref,
                  batch_ref, l1w_ref, l1b_ref, l2w_ref, l2b_ref, out_ref):
  deg = deg_ref[0, :, 0:1] + deg_ref[1, :, 0:1] + 1.0
  dinv = lax.rsqrt(deg)
  tot4 = s4_ref[0] + s4_ref[1] + g4_ref[...]
  h4 = jnp.tanh(dinv * tot4 + b4_ref[...])
  cat = jnp.concatenate(
      [h1_ref[...], h2_ref[...], h3_ref[...], h4[:, 0:1],
       jnp.zeros((NPAD, 31), jnp.float32)], axis=1)
  gids = lax.broadcasted_iota(jnp.int32, (NPAD, NG), 1)
  mt = (batch_ref[...] == gids).astype(jnp.float32)
  pooled = lax.dot_general(mt, cat, (((0,), (0,)), ((), ())),
                           preferred_element_type=jnp.float32)
  z = jnp.maximum(pooled @ l1w_ref[...] + l1b_ref[...], 0.0)
  logits = z @ l2w_ref[...] + l2b_ref[...]
  m = jnp.max(logits, axis=1, keepdims=True)
  lse = jnp.log(jnp.sum(jnp.exp(logits - m), axis=1, keepdims=True))
  out_ref[...] = logits - m - lse


_tc_g1 = pl.pallas_call(
    _tc_g1_body, out_shape=jax.ShapeDtypeStruct((NPAD, 32), jnp.float32))

_tc_layer_32 = pl.pallas_call(
    _tc_layer_body,
    out_shape=[jax.ShapeDtypeStruct((NPAD, 32), jnp.float32),
               jax.ShapeDtypeStruct((NPAD, 32), jnp.float32)])

_tc_layer_16 = pl.pallas_call(
    _tc_layer_body,
    out_shape=[jax.ShapeDtypeStruct((NPAD, 32), jnp.float32),
               jax.ShapeDtypeStruct((NPAD, 16), jnp.float32)])

_tc_head = pl.pallas_call(
    _tc_head_body, out_shape=jax.ShapeDtypeStruct((NG, 128), jnp.float32))


def kernel(x, edge_index, batch, W1, b1, W2, b2, W3, b3, W4, b4,
           lin1_W, lin1_b, lin2_W, lin2_b):
  src = edge_index[0].astype(jnp.int32)
  dst = edge_index[1].astype(jnp.int32)
  npad_e = EPAD - E
  src_p = jnp.concatenate([src, jnp.zeros((npad_e,), jnp.int32)])
  dst_p = jnp.concatenate([dst, jnp.full((npad_e,), N + 8, jnp.int32)])
  src_t = src_p.reshape(NW, NCHUNK, CHUNK)
  dst_t = dst_p.reshape(NW, NCHUNK, CHUNK)
  x_p = jnp.pad(x, ((0, NPAD - N), (0, 0)))
  batch_p = jnp.concatenate(
      [batch.astype(jnp.int32), jnp.full((NPAD - N,), NG, jnp.int32)])
  batch_2d = batch_p.reshape(NPAD, 1)
  zeros32 = jnp.zeros((NPAD, 32), jnp.float32)
  zeros16 = jnp.zeros((NPAD, 16), jnp.float32)
  ones16 = jnp.ones((CHUNK, 16), jnp.float32)
  w4_p = jnp.pad(W4, ((0, 0), (0, 15)))
  b4_p = jnp.pad(b4, (0, 15)).reshape(1, 16)
  l1w_p = jnp.pad(lin1_W, ((0, 128 - lin1_W.shape[0]), (0, 0)))
  l1b = lin1_b.reshape(1, 128)
  l2w_p = jnp.pad(lin2_W, ((0, 0), (0, 128 - lin2_W.shape[1])))
  l2b_p = jnp.concatenate(
      [lin2_b, jnp.full((128 - lin2_b.shape[0],), -1e30, jnp.float32)]
  ).reshape(1, 128)

  deg = _sc_deg(dst_t, zeros16, ones16)

  g1 = _tc_g1(x_p, W1, deg)
  s1 = _sc_seg(g1, src_t, dst_t, zeros32, 32)
  h1, g2 = _tc_layer_32(s1, g1, deg, b1.reshape(1, 32), W2)
  s2 = _sc_seg(g2, src_t, dst_t, zeros32, 32)
  h2, g3 = _tc_layer_32(s2, g2, deg, b2.reshape(1, 32), W3)
  s3 = _sc_seg(g3, src_t, dst_t, zeros32, 32)
  h3, g4 = _tc_layer_16(s3, g3, deg, b3.reshape(1, 32), w4_p)
  s4 = _sc_seg(g4, src_t, dst_t, zeros16, 16)

  out = _tc_head(s4, g4, deg, b4_p, h1, h2, h3, batch_2d,
                 l1w_p, l1b, l2w_p, l2b_p)
  return out[:, :10]


# speedup vs baseline: 1.0343x; 1.0343x over previous
import functools

import jax
import jax.numpy as jnp
from jax import lax
from jax.experimental import pallas as pl
from jax.experimental.pallas import tpu as pltpu
from jax.experimental.pallas import tpu_sc as plsc

N = 10000
NPAD = 10112
E = 320000
NG = 64
NC, NS = 2, 16
NW = NC * NS
CHUNK = 128
NCHUNK = 80
EPW = NCHUNK * CHUNK
EPAD = NW * EPW
ROWS_PER_TILE = NPAD // NS

@functools.lru_cache(maxsize=None)
def _get_mesh():
  return plsc.VectorSubcoreMesh(
      core_axis_name="c", subcore_axis_name="s", num_cores=NC, num_subcores=NS)


def _deg_body(dst_hbm, zeros_hbm, ones_hbm, out_hbm, didx, ones_v, acc, sem):
  c = lax.axis_index("c")
  s = lax.axis_index("s")
  w = s * NC + c
  rows = pl.ds(s * ROWS_PER_TILE, ROWS_PER_TILE)
  pltpu.sync_copy(zeros_hbm.at[rows], acc.at[rows])
  pltpu.sync_copy(dst_hbm.at[w], didx)
  pltpu.sync_copy(ones_hbm, ones_v)
  plsc.subcore_barrier()

  def body(j, carry):
    pltpu.sync_copy(ones_v, acc.at[didx.at[j]], add=True)
    return carry

  lax.fori_loop(0, NCHUNK, body, 0)
  plsc.subcore_barrier()
  pltpu.sync_copy(acc.at[rows], out_hbm.at[c, rows])


@functools.lru_cache(maxsize=None)
def _deg_call():
  return pl.kernel(
      _deg_body,
      out_type=jax.ShapeDtypeStruct((NC, NPAD, 16), jnp.float32),
      mesh=_get_mesh(),
      compiler_params=pltpu.CompilerParams(use_tc_tiling_on_sc=False),
      scratch_types=[
          pltpu.VMEM((NCHUNK, CHUNK), jnp.int32),
          pltpu.VMEM((CHUNK, 16), jnp.float32),
          pltpu.VMEM_SHARED((NPAD, 16), jnp.float32),
          pltpu.SemaphoreType.DMA,
      ],
  )


def _sc_deg(dst_t, zeros16, ones16):
  return _deg_call()(dst_t, zeros16, ones16)


NBUF = 4


def _seg_body(g_hbm, src_hbm, dst_hbm, zeros_hbm, out_hbm,
              sidx, didx, rows_bufs, acc, gsems):
  c = lax.axis_index("c")
  s = lax.axis_index("s")
  w = s * NC + c
  rows = pl.ds(s * ROWS_PER_TILE, ROWS_PER_TILE)
  pltpu.sync_copy(zeros_hbm.at[rows], acc.at[rows])
  pltpu.sync_copy(src_hbm.at[w], sidx)
  pltpu.sync_copy(dst_hbm.at[w], didx)
  plsc.subcore_barrier()

  for b in range(NBUF):
    pltpu.async_copy(g_hbm.at[sidx.at[b]], rows_bufs[b], gsems[b])

  def body(i, carry):
    j0 = NBUF * i
    for b in range(NBUF):
      pltpu.make_async_copy(g_hbm.at[sidx.at[0]], rows_bufs[b],
                            gsems[b]).wait()
      pltpu.sync_copy(rows_bufs[b], acc.at[didx.at[j0 + b]], add=True)
      pltpu.async_copy(g_hbm.at[sidx.at[j0 + NBUF + b]], rows_bufs[b],
                       gsems[b])
    return carry

  lax.fori_loop(0, NCHUNK // NBUF - 1, body, 0)
  j0 = NCHUNK - NBUF
  for b in range(NBUF):
    pltpu.make_async_copy(g_hbm.at[sidx.at[0]], rows_bufs[b],
                          gsems[b]).wait()
    pltpu.sync_copy(rows_bufs[b], acc.at[didx.at[j0 + b]], add=True)
  plsc.subcore_barrier()
  pltpu.sync_copy(acc.at[rows], out_hbm.at[c, rows])


@functools.lru_cache(maxsize=None)
def _seg_call(width):
  return pl.kernel(
      _seg_body,
      out_type=jax.ShapeDtypeStruct((NC, NPAD, width), jnp.float32),
      mesh=_get_mesh(),
      compiler_params=pltpu.CompilerParams(use_tc_tiling_on_sc=False),
      scratch_types=[
          pltpu.VMEM((NCHUNK, CHUNK), jnp.int32),
          pltpu.VMEM((NCHUNK, CHUNK), jnp.int32),
          [pltpu.VMEM((CHUNK, width), jnp.float32) for _ in range(NBUF)],
          pltpu.VMEM_SHARED((NPAD, width), jnp.float32),
          [pltpu.SemaphoreType.DMA for _ in range(NBUF)],
      ],
  )


def _sc_seg(g, src_t, dst_t, zeros, width):
  return _seg_call(width)(g, src_t, dst_t, zeros)


def _tc_g1_body(x_ref, w1_ref, deg_ref, g1_ref):
  deg = deg_ref[0, :, 0:1] + deg_ref[1, :, 0:1] + 1.0
  dinv = lax.rsqrt(deg)
  p = jnp.dot(x_ref[...], w1_ref[...], preferred_element_type=jnp.float32)
  g1_ref[...] = dinv * p


def _tc_layer_body(s_ref, g_ref, deg_ref, b_ref, wn_ref, h_ref, gn_ref):
  deg = deg_ref[0, :, 0:1] + deg_ref[1, :, 0:1] + 1.0
  dinv = lax.rsqrt(deg)
  tot = s_ref[0] + s_ref[1] + g_ref[...]
  h = jnp.tanh(dinv * tot + b_ref[...])
  row = lax.broadcasted_iota(jnp.int32, (NPAD, 1), 0)
  h = jnp.where(row < N, h, 0.0)
  h_ref[...] = h
  gn_ref[...] = dinv * jnp.dot(h, wn_ref[...],
                               preferred_element_type=jnp.float32)


def _tc_head_body(s4_ref, g4_ref, deg_ref, b4_ref, h1_ref, h2_ref, h3_ref,
                  batch_ref, l1w_ref, l1b_ref, l2w_ref, l2b_ref, out_ref):
  deg = deg_ref[0, :, 0:1] + deg_ref[1, :, 0:1] + 1.0
  dinv = lax.rsqrt(deg)
  tot4 = s4_ref[0] + s4_ref[1] + g4_ref[...]
  h4 = jnp.tanh(dinv * tot4 + b4_ref[...])
  cat = jnp.concatenate(
      [h1_ref[...], h2_ref[...], h3_ref[...], h4[:, 0:1],
       jnp.zeros((NPAD, 31), jnp.float32)], axis=1)
  gids = lax.broadcasted_iota(jnp.int32, (NPAD, NG), 1)
  mt = (batch_ref[...] == gids).astype(jnp.float32)
  pooled = lax.dot_general(mt, cat, (((0,), (0,)), ((), ())),
                           preferred_element_type=jnp.float32)
  z = jnp.maximum(pooled @ l1w_ref[...] + l1b_ref[...], 0.0)
  logits = z @ l2w_ref[...] + l2b_ref[...]
  m = jnp.max(logits, axis=1, keepdims=True)
  lse = jnp.log(jnp.sum(jnp.exp(logits - m), axis=1, keepdims=True))
  out_ref[...] = logits - m - lse


_tc_g1 = pl.pallas_call(
    _tc_g1_body, out_shape=jax.ShapeDtypeStruct((NPAD, 32), jnp.float32))

_tc_layer_32 = pl.pallas_call(
    _tc_layer_body,
    out_shape=[jax.ShapeDtypeStruct((NPAD, 32), jnp.float32),
               jax.ShapeDtypeStruct((NPAD, 32), jnp.float32)])

_tc_layer_16 = pl.pallas_call(
    _tc_layer_body,
    out_shape=[jax.ShapeDtypeStruct((NPAD, 32), jnp.float32),
               jax.ShapeDtypeStruct((NPAD, 16), jnp.float32)])

_tc_head = pl.pallas_call(
    _tc_head_body, out_shape=jax.ShapeDtypeStruct((NG, 128), jnp.float32))


def kernel(x, edge_index, batch, W1, b1, W2, b2, W3, b3, W4, b4,
           lin1_W, lin1_b, lin2_W, lin2_b):
  src = edge_index[0].astype(jnp.int32)
  dst = edge_index[1].astype(jnp.int32)
  npad_e = EPAD - E
  src_p = jnp.concatenate([src, jnp.zeros((npad_e,), jnp.int32)])
  dst_p = jnp.concatenate([dst, jnp.full((npad_e,), N + 8, jnp.int32)])
  src_t = src_p.reshape(NW, NCHUNK, CHUNK)
  dst_t = dst_p.reshape(NW, NCHUNK, CHUNK)
  x_p = jnp.pad(x, ((0, NPAD - N), (0, 0)))
  batch_p = jnp.concatenate(
      [batch.astype(jnp.int32), jnp.full((NPAD - N,), NG, jnp.int32)])
  batch_2d = batch_p.reshape(NPAD, 1)
  zeros32 = jnp.zeros((NPAD, 32), jnp.float32)
  zeros16 = jnp.zeros((NPAD, 16), jnp.float32)
  ones16 = jnp.ones((CHUNK, 16), jnp.float32)
  w4_p = jnp.pad(W4, ((0, 0), (0, 15)))
  b4_p = jnp.pad(b4, (0, 15)).reshape(1, 16)
  l1w_p = jnp.pad(lin1_W, ((0, 128 - lin1_W.shape[0]), (0, 0)))
  l1b = lin1_b.reshape(1, 128)
  l2w_p = jnp.pad(lin2_W, ((0, 0), (0, 128 - lin2_W.shape[1])))
  l2b_p = jnp.concatenate(
      [lin2_b, jnp.full((128 - lin2_b.shape[0],), -1e30, jnp.float32)]
  ).reshape(1, 128)

  deg = _sc_deg(dst_t, zeros16, ones16)

  g1 = _tc_g1(x_p, W1, deg)
  s1 = _sc_seg(g1, src_t, dst_t, zeros32, 32)
  h1, g2 = _tc_layer_32(s1, g1, deg, b1.reshape(1, 32), W2)
  s2 = _sc_seg(g2, src_t, dst_t, zeros32, 32)
  h2, g3 = _tc_layer_32(s2, g2, deg, b2.reshape(1, 32), W3)
  s3 = _sc_seg(g3, src_t, dst_t, zeros32, 32)
  h3, g4 = _tc_layer_16(s3, g3, deg, b3.reshape(1, 32), w4_p)
  s4 = _sc_seg(g4, src_t, dst_t, zeros16, 16)

  out = _tc_head(s4, g4, deg, b4_p, h1, h2, h3, batch_2d,
                 l1w_p, l1b, l2w_p, l2b_p)
  return out[:, :10]


# speedup vs baseline: 1.0389x; 1.0044x over previous
import functools

import jax
import jax.numpy as jnp
from jax import lax
from jax.experimental import pallas as pl
from jax.experimental.pallas import tpu as pltpu
from jax.experimental.pallas import tpu_sc as plsc

N = 10000
NPAD = 10112
E = 320000
NG = 64
NC, NS = 2, 16
NW = NC * NS
CHUNK = 128
NCHUNK = 80
EPW = NCHUNK * CHUNK
EPAD = NW * EPW
ROWS_PER_TILE = NPAD // NS

@functools.lru_cache(maxsize=None)
def _get_mesh():
  return plsc.VectorSubcoreMesh(
      core_axis_name="c", subcore_axis_name="s", num_cores=NC, num_subcores=NS)


def _deg_body(dst_hbm, zeros_hbm, ones_hbm, out_hbm, didx, ones_v, acc, sem):
  c = lax.axis_index("c")
  s = lax.axis_index("s")
  w = s * NC + c
  rows = pl.ds(s * ROWS_PER_TILE, ROWS_PER_TILE)
  pltpu.sync_copy(zeros_hbm.at[rows], acc.at[rows])
  pltpu.sync_copy(dst_hbm.at[w], didx)
  pltpu.sync_copy(ones_hbm, ones_v)
  plsc.subcore_barrier()

  def body(j, carry):
    pltpu.sync_copy(ones_v, acc.at[didx.at[j]], add=True)
    return carry

  lax.fori_loop(0, NCHUNK, body, 0)
  plsc.subcore_barrier()
  pltpu.sync_copy(acc.at[rows], out_hbm.at[c, rows])


@functools.lru_cache(maxsize=None)
def _deg_call():
  return pl.kernel(
      _deg_body,
      out_type=jax.ShapeDtypeStruct((NC, NPAD, 16), jnp.float32),
      mesh=_get_mesh(),
      compiler_params=pltpu.CompilerParams(use_tc_tiling_on_sc=False),
      scratch_types=[
          pltpu.VMEM((NCHUNK, CHUNK), jnp.int32),
          pltpu.VMEM((CHUNK, 16), jnp.float32),
          pltpu.VMEM_SHARED((NPAD, 16), jnp.float32),
          pltpu.SemaphoreType.DMA,
      ],
  )


def _sc_deg(dst_t, zeros16, ones16):
  return _deg_call()(dst_t, zeros16, ones16)


NBUF = 8


def _seg_body(g_hbm, src_hbm, dst_hbm, zeros_hbm, out_hbm,
              sidx, didx, rows_bufs, acc, gsems):
  c = lax.axis_index("c")
  s = lax.axis_index("s")
  w = s * NC + c
  rows = pl.ds(s * ROWS_PER_TILE, ROWS_PER_TILE)
  pltpu.sync_copy(zeros_hbm.at[rows], acc.at[rows])
  pltpu.sync_copy(src_hbm.at[w], sidx)
  pltpu.sync_copy(dst_hbm.at[w], didx)
  plsc.subcore_barrier()

  for b in range(NBUF):
    pltpu.async_copy(g_hbm.at[sidx.at[b]], rows_bufs[b], gsems[b])

  def body(i, carry):
    j0 = NBUF * i
    for b in range(NBUF):
      pltpu.make_async_copy(g_hbm.at[sidx.at[0]], rows_bufs[b],
                            gsems[b]).wait()
      pltpu.sync_copy(rows_bufs[b], acc.at[didx.at[j0 + b]], add=True)
      pltpu.async_copy(g_hbm.at[sidx.at[j0 + NBUF + b]], rows_bufs[b],
                       gsems[b])
    return carry

  lax.fori_loop(0, NCHUNK // NBUF - 1, body, 0)
  j0 = NCHUNK - NBUF
  for b in range(NBUF):
    pltpu.make_async_copy(g_hbm.at[sidx.at[0]], rows_bufs[b],
                          gsems[b]).wait()
    pltpu.sync_copy(rows_bufs[b], acc.at[didx.at[j0 + b]], add=True)
  plsc.subcore_barrier()
  pltpu.sync_copy(acc.at[rows], out_hbm.at[c, rows])


@functools.lru_cache(maxsize=None)
def _seg_call(width):
  return pl.kernel(
      _seg_body,
      out_type=jax.ShapeDtypeStruct((NC, NPAD, width), jnp.float32),
      mesh=_get_mesh(),
      compiler_params=pltpu.CompilerParams(use_tc_tiling_on_sc=False),
      scratch_types=[
          pltpu.VMEM((NCHUNK, CHUNK), jnp.int32),
          pltpu.VMEM((NCHUNK, CHUNK), jnp.int32),
          [pltpu.VMEM((CHUNK, width), jnp.float32) for _ in range(NBUF)],
          pltpu.VMEM_SHARED((NPAD, width), jnp.float32),
          [pltpu.SemaphoreType.DMA for _ in range(NBUF)],
      ],
  )


def _sc_seg(g, src_t, dst_t, zeros, width):
  return _seg_call(width)(g, src_t, dst_t, zeros)


def _tc_g1_body(x_ref, w1_ref, deg_ref, g1_ref):
  deg = deg_ref[0, :, 0:1] + deg_ref[1, :, 0:1] + 1.0
  dinv = lax.rsqrt(deg)
  p = jnp.dot(x_ref[...], w1_ref[...], preferred_element_type=jnp.float32)
  g1_ref[...] = dinv * p


def _tc_layer_body(s_ref, g_ref, deg_ref, b_ref, wn_ref, h_ref, gn_ref):
  deg = deg_ref[0, :, 0:1] + deg_ref[1, :, 0:1] + 1.0
  dinv = lax.rsqrt(deg)
  tot = s_ref[0] + s_ref[1] + g_ref[...]
  h = jnp.tanh(dinv * tot + b_ref[...])
  row = lax.broadcasted_iota(jnp.int32, (NPAD, 1), 0)
  h = jnp.where(row < N, h, 0.0)
  h_ref[...] = h
  gn_ref[...] = dinv * jnp.dot(h, wn_ref[...],
                               preferred_element_type=jnp.float32)


def _tc_head_body(s4_ref, g4_ref, deg_ref, b4_ref, h1_ref, h2_ref, h3_ref,
                  batch_ref, l1w_ref, l1b_ref, l2w_ref, l2b_ref, out_ref):
  deg = deg_ref[0, :, 0:1] + deg_ref[1, :, 0:1] + 1.0
  dinv = lax.rsqrt(deg)
  tot4 = s4_ref[0] + s4_ref[1] + g4_ref[...]
  h4 = jnp.tanh(dinv * tot4 + b4_ref[...])
  cat = jnp.concatenate(
      [h1_ref[...], h2_ref[...], h3_ref[...], h4[:, 0:1],
       jnp.zeros((NPAD, 31), jnp.float32)], axis=1)
  gids = lax.broadcasted_iota(jnp.int32, (NPAD, NG), 1)
  mt = (batch_ref[...] == gids).astype(jnp.float32)
  pooled = lax.dot_general(mt, cat, (((0,), (0,)), ((), ())),
                           preferred_element_type=jnp.float32)
  z = jnp.maximum(pooled @ l1w_ref[...] + l1b_ref[...], 0.0)
  logits = z @ l2w_ref[...] + l2b_ref[...]
  m = jnp.max(logits, axis=1, keepdims=True)
  lse = jnp.log(jnp.sum(jnp.exp(logits - m), axis=1, keepdims=True))
  out_ref[...] = logits - m - lse


_tc_g1 = pl.pallas_call(
    _tc_g1_body, out_shape=jax.ShapeDtypeStruct((NPAD, 32), jnp.float32))

_tc_layer_32 = pl.pallas_call(
    _tc_layer_body,
    out_shape=[jax.ShapeDtypeStruct((NPAD, 32), jnp.float32),
               jax.ShapeDtypeStruct((NPAD, 32), jnp.float32)])

_tc_layer_16 = pl.pallas_call(
    _tc_layer_body,
    out_shape=[jax.ShapeDtypeStruct((NPAD, 32), jnp.float32),
               jax.ShapeDtypeStruct((NPAD, 16), jnp.float32)])

_tc_head = pl.pallas_call(
    _tc_head_body, out_shape=jax.ShapeDtypeStruct((NG, 128), jnp.float32))


def kernel(x, edge_index, batch, W1, b1, W2, b2, W3, b3, W4, b4,
           lin1_W, lin1_b, lin2_W, lin2_b):
  src = edge_index[0].astype(jnp.int32)
  dst = edge_index[1].astype(jnp.int32)
  npad_e = EPAD - E
  src_p = jnp.concatenate([src, jnp.zeros((npad_e,), jnp.int32)])
  dst_p = jnp.concatenate([dst, jnp.full((npad_e,), N + 8, jnp.int32)])
  src_t = src_p.reshape(NW, NCHUNK, CHUNK)
  dst_t = dst_p.reshape(NW, NCHUNK, CHUNK)
  x_p = jnp.pad(x, ((0, NPAD - N), (0, 0)))
  batch_p = jnp.concatenate(
      [batch.astype(jnp.int32), jnp.full((NPAD - N,), NG, jnp.int32)])
  batch_2d = batch_p.reshape(NPAD, 1)
  zeros32 = jnp.zeros((NPAD, 32), jnp.float32)
  zeros16 = jnp.zeros((NPAD, 16), jnp.float32)
  ones16 = jnp.ones((CHUNK, 16), jnp.float32)
  w4_p = jnp.pad(W4, ((0, 0), (0, 15)))
  b4_p = jnp.pad(b4, (0, 15)).reshape(1, 16)
  l1w_p = jnp.pad(lin1_W, ((0, 128 - lin1_W.shape[0]), (0, 0)))
  l1b = lin1_b.reshape(1, 128)
  l2w_p = jnp.pad(lin2_W, ((0, 0), (0, 128 - lin2_W.shape[1])))
  l2b_p = jnp.concatenate(
      [lin2_b, jnp.full((128 - lin2_b.shape[0],), -1e30, jnp.float32)]
  ).reshape(1, 128)

  deg = _sc_deg(dst_t, zeros16, ones16)

  g1 = _tc_g1(x_p, W1, deg)
  s1 = _sc_seg(g1, src_t, dst_t, zeros32, 32)
  h1, g2 = _tc_layer_32(s1, g1, deg, b1.reshape(1, 32), W2)
  s2 = _sc_seg(g2, src_t, dst_t, zeros32, 32)
  h2, g3 = _tc_layer_32(s2, g2, deg, b2.reshape(1, 32), W3)
  s3 = _sc_seg(g3, src_t, dst_t, zeros32, 32)
  h3, g4 = _tc_layer_16(s3, g3, deg, b3.reshape(1, 32), w4_p)
  s4 = _sc_seg(g4, src_t, dst_t, zeros16, 16)

  out = _tc_head(s4, g4, deg, b4_p, h1, h2, h3, batch_2d,
                 l1w_p, l1b, l2w_p, l2b_p)
  return out[:, :10]
